# baseline (speedup 1.0000x reference)
import functools

import jax
import jax.numpy as jnp
from jax import lax
from jax.experimental import pallas as pl
from jax.experimental.pallas import tpu as pltpu
from jax.experimental.pallas import tpu_sc as plsc

NC = 2
NS = 16
NW = NC * NS
CHUNK = 128


def _sc_gather(gu3, gi3, utR, itR, b_per_w):
    n_chunks = b_per_w // CHUNK
    B = NW * b_per_w
    mesh = plsc.VectorSubcoreMesh(core_axis_name="c", subcore_axis_name="s")
    out_t = (
        jax.ShapeDtypeStruct((B, 128), jnp.float32),
        jax.ShapeDtypeStruct((B, 128), jnp.float32),
    )

    @functools.partial(
        pl.kernel,
        mesh=mesh,
        out_type=out_t,
        scratch_types=[
            pltpu.VMEM((n_chunks, CHUNK), jnp.int32),
            pltpu.VMEM((n_chunks, CHUNK), jnp.int32),
            pltpu.VMEM((b_per_w, 128), jnp.float32),
            pltpu.SemaphoreType.DMA,
        ],
        compiler_params=pltpu.CompilerParams(use_tc_tiling_on_sc=True),
    )
    def k(gu_hbm, gi_hbm, ut_hbm, it_hbm, x4u_out, x4i_out,
          uidx_v, iidx_v, loc_v, sem):
        wid = lax.axis_index("s") * NC + lax.axis_index("c")
        base = wid * b_per_w
        pltpu.sync_copy(gu_hbm.at[wid], uidx_v)
        pltpu.sync_copy(gi_hbm.at[wid], iidx_v)
        copies = []
        for j in range(n_chunks):
            copies.append(pltpu.async_copy(
                ut_hbm.at[uidx_v.at[j]], loc_v.at[pl.ds(j * CHUNK, CHUNK)], sem))
        for c in copies:
            c.wait()
        pltpu.sync_copy(loc_v, x4u_out.at[pl.ds(base, b_per_w)])
        copies = []
        for j in range(n_chunks):
            copies.append(pltpu.async_copy(
                it_hbm.at[iidx_v.at[j]], loc_v.at[pl.ds(j * CHUNK, CHUNK)], sem))
        for c in copies:
            c.wait()
        pltpu.sync_copy(loc_v, x4i_out.at[pl.ds(base, b_per_w)])

    return k(gu3, gi3, utR, itR)


def _mlp_body(x4u_ref, x4i_ref, qu_ref, qi_ref, w1a_ref, w1b_ref, b1_ref,
              w2t_ref, b2_ref, w3_ref, b3_ref, out_ref):
    lane_q = lax.broadcasted_iota(jnp.int32, x4u_ref.shape, 1) >> 5
    xu = x4u_ref[...] * (lane_q == qu_ref[...]).astype(jnp.float32)
    xi = x4i_ref[...] * (lane_q == qi_ref[...]).astype(jnp.float32)
    h1 = (jnp.dot(xu, w1a_ref[...], preferred_element_type=jnp.float32)
          + jnp.dot(xi, w1b_ref[...], preferred_element_type=jnp.float32)
          + b1_ref[...])
    h1 = jnp.maximum(h1, 0.0)
    h2 = jnp.dot(h1, w2t_ref[...], preferred_element_type=jnp.float32) + b2_ref[...]
    h2 = jnp.maximum(h2, 0.0)
    logit = jnp.sum(h2 * w3_ref[...], axis=1, keepdims=True) + b3_ref[0, 0]
    out_ref[...] = jax.nn.sigmoid(logit)


def _mlp(x4u, x4i, qu, qi, W1, b1, W2, b2, W3, b3, interpret=False):
    B = x4u.shape[0]
    emb = W1.shape[1] // 2
    n1 = W1.shape[0]
    n2 = W2.shape[0]
    w1t = W1.T
    w1a4 = jnp.concatenate([w1t[:emb]] * 4, axis=0)
    w1b4 = jnp.concatenate([w1t[emb:]] * 4, axis=0)
    w2t = W2.T
    b1r = b1.reshape(1, -1)
    b2r = b2.reshape(1, -1)
    w3r = W3.reshape(1, -1)
    b3r = b3.reshape(1, 1)

    bb = 4096
    grid = (B // bb,)
    fixed = lambda shape: pl.BlockSpec(shape, lambda j: (0, 0))
    out = pl.pallas_call(
        _mlp_body,
        grid=grid,
        in_specs=[
            pl.BlockSpec((bb, 128), lambda j: (j, 0)),
            pl.BlockSpec((bb, 128), lambda j: (j, 0)),
            pl.BlockSpec((bb, 1), lambda j: (j, 0)),
            pl.BlockSpec((bb, 1), lambda j: (j, 0)),
            fixed((128, n1)),
            fixed((128, n1)),
            fixed((1, n1)),
            fixed((n1, n2)),
            fixed((1, n2)),
            fixed((1, n2)),
            fixed((1, 1)),
        ],
        out_specs=pl.BlockSpec((bb, 1), lambda j: (j, 0)),
        out_shape=jax.ShapeDtypeStruct((B, 1), jnp.float32),
        interpret=interpret,
    )(x4u, x4i, qu, qi, w1a4, w1b4, b1r, w2t, b2r, w3r, b3r)
    return jnp.squeeze(out, axis=-1)


def kernel(u, i, user_table, item_table, W1, b1, W2, b2, W3, b3):
    B = u.shape[0]
    V = user_table.shape[0]
    b_per_w = B // NW
    n_chunks = b_per_w // CHUNK
    u32 = u.astype(jnp.int32)
    i32 = i.astype(jnp.int32)
    gu3 = (u32 >> 2).reshape(NW, n_chunks, CHUNK)
    gi3 = (i32 >> 2).reshape(NW, n_chunks, CHUNK)
    utR = user_table.reshape(V // 4, 128)
    itR = item_table.reshape(V // 4, 128)
    x4u, x4i = _sc_gather(gu3, gi3, utR, itR, b_per_w)
    qu = (u32 & 3).reshape(B, 1)
    qi = (i32 & 3).reshape(B, 1)
    return _mlp(x4u, x4i, qu, qi, W1, b1, W2, b2, W3, b3)


# speedup vs baseline: 1.1127x; 1.1127x over previous
import functools

import jax
import jax.numpy as jnp
from jax import lax
from jax.experimental import pallas as pl
from jax.experimental.pallas import tpu as pltpu
from jax.experimental.pallas import tpu_sc as plsc

NC = 2
NS = 16
NW = NC * NS
CHUNK = 128


def _sc_gather(gu3, gi3, utR, itR, b_per_w):
    n_chunks = b_per_w // CHUNK
    B = NW * b_per_w
    mesh = plsc.VectorSubcoreMesh(core_axis_name="c", subcore_axis_name="s")
    out_t = (
        jax.ShapeDtypeStruct((B, 128), jnp.float32),
        jax.ShapeDtypeStruct((B, 128), jnp.float32),
    )

    @functools.partial(
        pl.kernel,
        mesh=mesh,
        out_type=out_t,
        scratch_types=[
            pltpu.VMEM((n_chunks, CHUNK), jnp.int32),
            pltpu.VMEM((n_chunks, CHUNK), jnp.int32),
            pltpu.VMEM((b_per_w, 128), jnp.float32),
            pltpu.SemaphoreType.DMA,
        ],
        compiler_params=pltpu.CompilerParams(use_tc_tiling_on_sc=True),
    )
    def k(gu_hbm, gi_hbm, ut_hbm, it_hbm, x4u_out, x4i_out,
          uidx_v, iidx_v, loc_v, sem):
        wid = lax.axis_index("s") * NC + lax.axis_index("c")
        base = wid * b_per_w
        pltpu.sync_copy(gu_hbm.at[wid], uidx_v)
        pltpu.sync_copy(gi_hbm.at[wid], iidx_v)
        copies = []
        for j in range(n_chunks):
            copies.append(pltpu.async_copy(
                ut_hbm.at[uidx_v.at[j]], loc_v.at[pl.ds(j * CHUNK, CHUNK)], sem))
        for c in copies:
            c.wait()
        pltpu.sync_copy(loc_v, x4u_out.at[pl.ds(base, b_per_w)])
        copies = []
        for j in range(n_chunks):
            copies.append(pltpu.async_copy(
                it_hbm.at[iidx_v.at[j]], loc_v.at[pl.ds(j * CHUNK, CHUNK)], sem))
        for c in copies:
            c.wait()
        pltpu.sync_copy(loc_v, x4i_out.at[pl.ds(base, b_per_w)])

    return k(gu3, gi3, utR, itR)


def _xpose_body(xt_ref, out_ref):
    w = xt_ref[...].T
    c, emb = w.shape
    w3 = w.reshape(c // 4, 4, emb)
    out_ref[...] = jnp.concatenate([w3[:, k, :] for k in range(4)], axis=1)


def _xpose(tT, interpret=False):
    emb, V = tT.shape
    C = 8192
    grid = (pl.cdiv(V, C),)
    return pl.pallas_call(
        _xpose_body,
        grid=grid,
        in_specs=[pl.BlockSpec((emb, C), lambda j: (0, j))],
        out_specs=pl.BlockSpec((C // 4, 4 * emb), lambda j: (j, 0)),
        out_shape=jax.ShapeDtypeStruct((V // 4, 4 * emb), jnp.float32),
        interpret=interpret,
    )(tT)


def _mlp_body(x4u_ref, x4i_ref, qu_ref, qi_ref, w1a_ref, w1b_ref, b1_ref,
              w2t_ref, b2_ref, w3_ref, b3_ref, out_ref):
    lane_q = lax.broadcasted_iota(jnp.int32, x4u_ref.shape, 1) >> 5
    xu = x4u_ref[...] * (lane_q == qu_ref[...]).astype(jnp.float32)
    xi = x4i_ref[...] * (lane_q == qi_ref[...]).astype(jnp.float32)
    h1 = (jnp.dot(xu, w1a_ref[...], preferred_element_type=jnp.float32)
          + jnp.dot(xi, w1b_ref[...], preferred_element_type=jnp.float32)
          + b1_ref[...])
    h1 = jnp.maximum(h1, 0.0)
    h2 = jnp.dot(h1, w2t_ref[...], preferred_element_type=jnp.float32) + b2_ref[...]
    h2 = jnp.maximum(h2, 0.0)
    logit = jnp.sum(h2 * w3_ref[...], axis=1, keepdims=True) + b3_ref[0, 0]
    out_ref[...] = jax.nn.sigmoid(logit)


def _mlp(x4u, x4i, qu, qi, W1, b1, W2, b2, W3, b3, interpret=False):
    B = x4u.shape[0]
    emb = W1.shape[1] // 2
    n1 = W1.shape[0]
    n2 = W2.shape[0]
    w1t = W1.T
    w1a4 = jnp.concatenate([w1t[:emb]] * 4, axis=0)
    w1b4 = jnp.concatenate([w1t[emb:]] * 4, axis=0)
    w2t = W2.T
    b1r = b1.reshape(1, -1)
    b2r = b2.reshape(1, -1)
    w3r = W3.reshape(1, -1)
    b3r = b3.reshape(1, 1)

    bb = 4096
    grid = (B // bb,)
    fixed = lambda shape: pl.BlockSpec(shape, lambda j: (0, 0))
    out = pl.pallas_call(
        _mlp_body,
        grid=grid,
        in_specs=[
            pl.BlockSpec((bb, 128), lambda j: (j, 0)),
            pl.BlockSpec((bb, 128), lambda j: (j, 0)),
            pl.BlockSpec((bb, 1), lambda j: (j, 0)),
            pl.BlockSpec((bb, 1), lambda j: (j, 0)),
            fixed((128, n1)),
            fixed((128, n1)),
            fixed((1, n1)),
            fixed((n1, n2)),
            fixed((1, n2)),
            fixed((1, n2)),
            fixed((1, 1)),
        ],
        out_specs=pl.BlockSpec((bb, 1), lambda j: (j, 0)),
        out_shape=jax.ShapeDtypeStruct((B, 1), jnp.float32),
        interpret=interpret,
    )(x4u, x4i, qu, qi, w1a4, w1b4, b1r, w2t, b2r, w3r, b3r)
    return jnp.squeeze(out, axis=-1)


def kernel(u, i, user_table, item_table, W1, b1, W2, b2, W3, b3):
    B = u.shape[0]
    V = user_table.shape[0]
    b_per_w = B // NW
    n_chunks = b_per_w // CHUNK
    u32 = u.astype(jnp.int32)
    i32 = i.astype(jnp.int32)
    gu3 = (u32 >> 2).reshape(NW, n_chunks, CHUNK)
    gi3 = (i32 >> 2).reshape(NW, n_chunks, CHUNK)
    utR = _xpose(user_table.T)
    itR = _xpose(item_table.T)
    x4u, x4i = _sc_gather(gu3, gi3, utR, itR, b_per_w)
    qu = (u32 & 3).reshape(B, 1)
    qi = (i32 & 3).reshape(B, 1)
    return _mlp(x4u, x4i, qu, qi, W1, b1, W2, b2, W3, b3)


# speedup vs baseline: 1.2010x; 1.0793x over previous
import functools

import jax
import jax.numpy as jnp
from jax import lax
from jax.experimental import pallas as pl
from jax.experimental.pallas import tpu as pltpu
from jax.experimental.pallas import tpu_sc as plsc

NC = 2
NS = 16
NW = NC * NS
CHUNK = 128


def _sc_gather(gu3, gi3, utR, itR, b_per_w):
    n_chunks = b_per_w // CHUNK
    B = NW * b_per_w
    mesh = plsc.VectorSubcoreMesh(core_axis_name="c", subcore_axis_name="s")
    out_t = (
        jax.ShapeDtypeStruct((B, 128), jnp.float32),
        jax.ShapeDtypeStruct((B, 128), jnp.float32),
    )

    @functools.partial(
        pl.kernel,
        mesh=mesh,
        out_type=out_t,
        scratch_types=[
            pltpu.VMEM((n_chunks, CHUNK), jnp.int32),
            pltpu.VMEM((n_chunks, CHUNK), jnp.int32),
            pltpu.VMEM((b_per_w, 128), jnp.float32),
            pltpu.SemaphoreType.DMA,
        ],
        compiler_params=pltpu.CompilerParams(use_tc_tiling_on_sc=True),
    )
    def k(gu_hbm, gi_hbm, ut_hbm, it_hbm, x4u_out, x4i_out,
          uidx_v, iidx_v, loc_v, sem):
        wid = lax.axis_index("s") * NC + lax.axis_index("c")
        base = wid * b_per_w
        pltpu.sync_copy(gu_hbm.at[wid], uidx_v)
        pltpu.sync_copy(gi_hbm.at[wid], iidx_v)
        copies = []
        for j in range(n_chunks):
            copies.append(pltpu.async_copy(
                ut_hbm.at[uidx_v.at[j]], loc_v.at[pl.ds(j * CHUNK, CHUNK)], sem))
        for c in copies:
            c.wait()
        pltpu.sync_copy(loc_v, x4u_out.at[pl.ds(base, b_per_w)])
        copies = []
        for j in range(n_chunks):
            copies.append(pltpu.async_copy(
                it_hbm.at[iidx_v.at[j]], loc_v.at[pl.ds(j * CHUNK, CHUNK)], sem))
        for c in copies:
            c.wait()
        pltpu.sync_copy(loc_v, x4i_out.at[pl.ds(base, b_per_w)])

    return k(gu3, gi3, utR, itR)


def _xpose_body(xt_ref, out_ref):
    x = xt_ref[...]
    emb, c = x.shape
    w = x.T
    w3 = w.reshape(c // 4, 4, emb)
    out_ref[...] = jnp.concatenate([w3[:, k, :] for k in range(4)], axis=1)


def _xpose(tT, interpret=False):
    emb, V = tT.shape
    C = 16384
    grid = (pl.cdiv(V, C),)
    return pl.pallas_call(
        _xpose_body,
        grid=grid,
        in_specs=[pl.BlockSpec((emb, C), lambda j: (0, j))],
        out_specs=pl.BlockSpec((C // 4, 4 * emb), lambda j: (j, 0)),
        out_shape=jax.ShapeDtypeStruct((V // 4, 4 * emb), jnp.float32),
        compiler_params=pltpu.CompilerParams(fuse_transposed_lhs_in_matmul=True),
        interpret=interpret,
    )(tT)


def _mlp_body(x4u_ref, x4i_ref, qu_ref, qi_ref, w1a_ref, w1b_ref, b1_ref,
              w2t_ref, b2_ref, w3_ref, b3_ref, out_ref):
    lane_q = lax.broadcasted_iota(jnp.int32, x4u_ref.shape, 1) >> 5
    xu = x4u_ref[...] * (lane_q == qu_ref[...]).astype(jnp.float32)
    xi = x4i_ref[...] * (lane_q == qi_ref[...]).astype(jnp.float32)
    h1 = (jnp.dot(xu, w1a_ref[...], preferred_element_type=jnp.float32)
          + jnp.dot(xi, w1b_ref[...], preferred_element_type=jnp.float32)
          + b1_ref[...])
    h1 = jnp.maximum(h1, 0.0)
    h2 = jnp.dot(h1, w2t_ref[...], preferred_element_type=jnp.float32) + b2_ref[...]
    h2 = jnp.maximum(h2, 0.0)
    logit = jnp.sum(h2 * w3_ref[...], axis=1, keepdims=True) + b3_ref[0, 0]
    out_ref[...] = jax.nn.sigmoid(logit)


def _mlp(x4u, x4i, qu, qi, W1, b1, W2, b2, W3, b3, interpret=False):
    B = x4u.shape[0]
    emb = W1.shape[1] // 2
    n1 = W1.shape[0]
    n2 = W2.shape[0]
    w1t = W1.T
    w1a4 = jnp.concatenate([w1t[:emb]] * 4, axis=0)
    w1b4 = jnp.concatenate([w1t[emb:]] * 4, axis=0)
    w2t = W2.T
    b1r = b1.reshape(1, -1)
    b2r = b2.reshape(1, -1)
    w3r = W3.reshape(1, -1)
    b3r = b3.reshape(1, 1)

    bb = 4096
    grid = (B // bb,)
    fixed = lambda shape: pl.BlockSpec(shape, lambda j: (0, 0))
    out = pl.pallas_call(
        _mlp_body,
        grid=grid,
        in_specs=[
            pl.BlockSpec((bb, 128), lambda j: (j, 0)),
            pl.BlockSpec((bb, 128), lambda j: (j, 0)),
            pl.BlockSpec((bb, 1), lambda j: (j, 0)),
            pl.BlockSpec((bb, 1), lambda j: (j, 0)),
            fixed((128, n1)),
            fixed((128, n1)),
            fixed((1, n1)),
            fixed((n1, n2)),
            fixed((1, n2)),
            fixed((1, n2)),
            fixed((1, 1)),
        ],
        out_specs=pl.BlockSpec((bb, 1), lambda j: (j, 0)),
        out_shape=jax.ShapeDtypeStruct((B, 1), jnp.float32),
        interpret=interpret,
    )(x4u, x4i, qu, qi, w1a4, w1b4, b1r, w2t, b2r, w3r, b3r)
    return jnp.squeeze(out, axis=-1)


def kernel(u, i, user_table, item_table, W1, b1, W2, b2, W3, b3):
    B = u.shape[0]
    V = user_table.shape[0]
    b_per_w = B // NW
    n_chunks = b_per_w // CHUNK
    u32 = u.astype(jnp.int32)
    i32 = i.astype(jnp.int32)
    gu3 = (u32 >> 2).reshape(NW, n_chunks, CHUNK)
    gi3 = (i32 >> 2).reshape(NW, n_chunks, CHUNK)
    utR = _xpose(user_table.T)
    itR = item_table.reshape(V // 4, 128)
    x4u, x4i = _sc_gather(gu3, gi3, utR, itR, b_per_w)
    qu = (u32 & 3).reshape(B, 1)
    qi = (i32 & 3).reshape(B, 1)
    return _mlp(x4u, x4i, qu, qi, W1, b1, W2, b2, W3, b3)


# speedup vs baseline: 1.2072x; 1.0052x over previous
import functools

import jax
import jax.numpy as jnp
from jax import lax
from jax.experimental import pallas as pl
from jax.experimental.pallas import tpu as pltpu
from jax.experimental.pallas import tpu_sc as plsc

NC = 2
NS = 16
NW = NC * NS
CHUNK = 128


def _sc_gather(g3, tR, b_per_w):
    n_chunks = b_per_w // CHUNK
    B = NW * b_per_w
    mesh = plsc.VectorSubcoreMesh(core_axis_name="c", subcore_axis_name="s")

    @functools.partial(
        pl.kernel,
        mesh=mesh,
        out_type=jax.ShapeDtypeStruct((B, 128), jnp.float32),
        scratch_types=[
            pltpu.VMEM((n_chunks, CHUNK), jnp.int32),
            pltpu.VMEM((b_per_w, 128), jnp.float32),
            pltpu.SemaphoreType.DMA,
        ],
        compiler_params=pltpu.CompilerParams(use_tc_tiling_on_sc=True),
    )
    def k(g_hbm, t_hbm, x4_out, idx_v, loc_v, sem):
        wid = lax.axis_index("s") * NC + lax.axis_index("c")
        base = wid * b_per_w
        pltpu.sync_copy(g_hbm.at[wid], idx_v)
        copies = []
        for j in range(n_chunks):
            copies.append(pltpu.async_copy(
                t_hbm.at[idx_v.at[j]], loc_v.at[pl.ds(j * CHUNK, CHUNK)], sem))
        for c in copies:
            c.wait()
        pltpu.sync_copy(loc_v, x4_out.at[pl.ds(base, b_per_w)])

    return k(g3, tR)


def _xpose_body(xt_ref, out_ref):
    x = xt_ref[...]
    emb, c = x.shape
    w = x.T
    w3 = w.reshape(c // 4, 4, emb)
    out_ref[...] = jnp.concatenate([w3[:, k, :] for k in range(4)], axis=1)


def _xpose(tT, interpret=False):
    emb, V = tT.shape
    C = 32768
    grid = (pl.cdiv(V, C),)
    return pl.pallas_call(
        _xpose_body,
        grid=grid,
        in_specs=[pl.BlockSpec((emb, C), lambda j: (0, j))],
        out_specs=pl.BlockSpec((C // 4, 4 * emb), lambda j: (j, 0)),
        out_shape=jax.ShapeDtypeStruct((V // 4, 4 * emb), jnp.float32),
        compiler_params=pltpu.CompilerParams(fuse_transposed_lhs_in_matmul=True),
        interpret=interpret,
    )(tT)


def _mlp_body(x4u_ref, x4i_ref, qu_ref, qi_ref, w1a_ref, w1b_ref, b1_ref,
              w2t_ref, b2_ref, w3_ref, b3_ref, out_ref):
    lane_q = lax.broadcasted_iota(jnp.int32, x4u_ref.shape, 1) >> 5
    xu = x4u_ref[...] * (lane_q == qu_ref[...]).astype(jnp.float32)
    xi = x4i_ref[...] * (lane_q == qi_ref[...]).astype(jnp.float32)
    h1 = (jnp.dot(xu, w1a_ref[...], preferred_element_type=jnp.float32)
          + jnp.dot(xi, w1b_ref[...], preferred_element_type=jnp.float32)
          + b1_ref[...])
    h1 = jnp.maximum(h1, 0.0)
    h2 = jnp.dot(h1, w2t_ref[...], preferred_element_type=jnp.float32) + b2_ref[...]
    h2 = jnp.maximum(h2, 0.0)
    logit = jnp.sum(h2 * w3_ref[...], axis=1, keepdims=True) + b3_ref[0, 0]
    out_ref[...] = jax.nn.sigmoid(logit)


def _mlp(x4u, x4i, qu, qi, W1, b1, W2, b2, W3, b3, interpret=False):
    B = x4u.shape[0]
    emb = W1.shape[1] // 2
    n1 = W1.shape[0]
    n2 = W2.shape[0]
    w1t = W1.T
    w1a4 = jnp.concatenate([w1t[:emb]] * 4, axis=0)
    w1b4 = jnp.concatenate([w1t[emb:]] * 4, axis=0)
    w2t = W2.T
    b1r = b1.reshape(1, -1)
    b2r = b2.reshape(1, -1)
    w3r = W3.reshape(1, -1)
    b3r = b3.reshape(1, 1)

    bb = 4096
    grid = (B // bb,)
    fixed = lambda shape: pl.BlockSpec(shape, lambda j: (0, 0))
    out = pl.pallas_call(
        _mlp_body,
        grid=grid,
        in_specs=[
            pl.BlockSpec((bb, 128), lambda j: (j, 0)),
            pl.BlockSpec((bb, 128), lambda j: (j, 0)),
            pl.BlockSpec((bb, 1), lambda j: (j, 0)),
            pl.BlockSpec((bb, 1), lambda j: (j, 0)),
            fixed((128, n1)),
            fixed((128, n1)),
            fixed((1, n1)),
            fixed((n1, n2)),
            fixed((1, n2)),
            fixed((1, n2)),
            fixed((1, 1)),
        ],
        out_specs=pl.BlockSpec((bb, 1), lambda j: (j, 0)),
        out_shape=jax.ShapeDtypeStruct((B, 1), jnp.float32),
        interpret=interpret,
    )(x4u, x4i, qu, qi, w1a4, w1b4, b1r, w2t, b2r, w3r, b3r)
    return jnp.squeeze(out, axis=-1)


def kernel(u, i, user_table, item_table, W1, b1, W2, b2, W3, b3):
    B = u.shape[0]
    V = user_table.shape[0]
    b_per_w = B // NW
    n_chunks = b_per_w // CHUNK
    u32 = u.astype(jnp.int32)
    i32 = i.astype(jnp.int32)
    gu3 = (u32 >> 2).reshape(NW, n_chunks, CHUNK)
    gi3 = (i32 >> 2).reshape(NW, n_chunks, CHUNK)
    utR = _xpose(user_table.T)
    itR = item_table.reshape(V // 4, 128)
    x4u = _sc_gather(gu3, utR, b_per_w)
    x4i = _sc_gather(gi3, itR, b_per_w)
    qu = (u32 & 3).reshape(B, 1)
    qi = (i32 & 3).reshape(B, 1)
    return _mlp(x4u, x4i, qu, qi, W1, b1, W2, b2, W3, b3)


# speedup vs baseline: 1.2275x; 1.0168x over previous
import functools

import jax
import jax.numpy as jnp
from jax import lax
from jax.experimental import pallas as pl
from jax.experimental.pallas import tpu as pltpu
from jax.experimental.pallas import tpu_sc as plsc

NC = 2
NS = 16
NW = NC * NS
CHUNK = 128


def _sc_gather(g3, tR, b_per_w):
    n_chunks = b_per_w // CHUNK
    B = NW * b_per_w
    mesh = plsc.VectorSubcoreMesh(core_axis_name="c", subcore_axis_name="s")

    @functools.partial(
        pl.kernel,
        mesh=mesh,
        out_type=jax.ShapeDtypeStruct((B, 128), jnp.float32),
        scratch_types=[
            pltpu.VMEM((n_chunks, CHUNK), jnp.int32),
            pltpu.VMEM((b_per_w, 128), jnp.float32),
            pltpu.SemaphoreType.DMA,
        ],
        compiler_params=pltpu.CompilerParams(use_tc_tiling_on_sc=True),
    )
    def k(g_hbm, t_hbm, x4_out, idx_v, loc_v, sem):
        wid = lax.axis_index("s") * NC + lax.axis_index("c")
        base = wid * b_per_w
        pltpu.sync_copy(g_hbm.at[wid], idx_v)
        copies = []
        for j in range(n_chunks):
            copies.append(pltpu.async_copy(
                t_hbm.at[idx_v.at[j]], loc_v.at[pl.ds(j * CHUNK, CHUNK)], sem))
        for c in copies:
            c.wait()
        pltpu.sync_copy(loc_v, x4_out.at[pl.ds(base, b_per_w)])

    return k(g3, tR)


def _xpose_body(xt_ref, out_ref):
    x = xt_ref[...]
    emb, c = x.shape
    w = x.T
    w3 = w.reshape(c // 4, 4, emb)
    out_ref[...] = jnp.concatenate([w3[:, k, :] for k in range(4)], axis=1)


def _xpose(tT, interpret=False):
    emb, V = tT.shape
    C = 32768
    grid = (pl.cdiv(V, C),)
    return pl.pallas_call(
        _xpose_body,
        grid=grid,
        in_specs=[pl.BlockSpec((emb, C), lambda j: (0, j))],
        out_specs=pl.BlockSpec((C // 4, 4 * emb), lambda j: (j, 0)),
        out_shape=jax.ShapeDtypeStruct((V // 4, 4 * emb), jnp.float32),
        compiler_params=pltpu.CompilerParams(fuse_transposed_lhs_in_matmul=True),
        interpret=interpret,
    )(tT)


def _mlp_body(x4u_ref, x4i_ref, qu_ref, qi_ref, w1a_ref, w1b_ref, b1_ref,
              w2t_ref, b2_ref, w3_ref, b3_ref, out_ref):
    lane_q = lax.broadcasted_iota(jnp.int32, x4u_ref.shape, 1) >> 5
    qu = qu_ref[...].reshape(-1, 1)
    qi = qi_ref[...].reshape(-1, 1)
    xu = x4u_ref[...] * (lane_q == qu).astype(jnp.float32)
    xi = x4i_ref[...] * (lane_q == qi).astype(jnp.float32)
    h1 = (jnp.dot(xu, w1a_ref[...], preferred_element_type=jnp.float32)
          + jnp.dot(xi, w1b_ref[...], preferred_element_type=jnp.float32)
          + b1_ref[...])
    h1 = jnp.maximum(h1, 0.0)
    h2 = jnp.dot(h1, w2t_ref[...], preferred_element_type=jnp.float32) + b2_ref[...]
    h2 = jnp.maximum(h2, 0.0)
    logit = jnp.sum(h2 * w3_ref[...], axis=1, keepdims=True) + b3_ref[0, 0]
    out_ref[...] = jax.nn.sigmoid(logit)


def _mlp(x4u, x4i, qu, qi, W1, b1, W2, b2, W3, b3, interpret=False):
    B = x4u.shape[0]
    emb = W1.shape[1] // 2
    n1 = W1.shape[0]
    n2 = W2.shape[0]
    w1t = W1.T
    w1a4 = jnp.concatenate([w1t[:emb]] * 4, axis=0)
    w1b4 = jnp.concatenate([w1t[emb:]] * 4, axis=0)
    w2t = W2.T
    b1r = b1.reshape(1, -1)
    b2r = b2.reshape(1, -1)
    w3r = W3.reshape(1, -1)
    b3r = b3.reshape(1, 1)

    bb = 4096
    grid = (B // bb,)
    fixed = lambda shape: pl.BlockSpec(shape, lambda j: (0, 0))
    out = pl.pallas_call(
        _mlp_body,
        grid=grid,
        in_specs=[
            pl.BlockSpec((bb, 128), lambda j: (j, 0)),
            pl.BlockSpec((bb, 128), lambda j: (j, 0)),
            pl.BlockSpec((bb,), lambda j: (j,)),
            pl.BlockSpec((bb,), lambda j: (j,)),
            fixed((128, n1)),
            fixed((128, n1)),
            fixed((1, n1)),
            fixed((n1, n2)),
            fixed((1, n2)),
            fixed((1, n2)),
            fixed((1, 1)),
        ],
        out_specs=pl.BlockSpec((bb, 1), lambda j: (j, 0)),
        out_shape=jax.ShapeDtypeStruct((B, 1), jnp.float32),
        interpret=interpret,
    )(x4u, x4i, qu, qi, w1a4, w1b4, b1r, w2t, b2r, w3r, b3r)
    return jnp.squeeze(out, axis=-1)


def kernel(u, i, user_table, item_table, W1, b1, W2, b2, W3, b3):
    B = u.shape[0]
    V = user_table.shape[0]
    b_per_w = B // NW
    n_chunks = b_per_w // CHUNK
    u32 = u.astype(jnp.int32)
    i32 = i.astype(jnp.int32)
    gu3 = (u32 >> 2).reshape(NW, n_chunks, CHUNK)
    gi3 = (i32 >> 2).reshape(NW, n_chunks, CHUNK)
    utR = _xpose(user_table.T)
    itR = item_table.reshape(V // 4, 128)
    x4u = _sc_gather(gu3, utR, b_per_w)
    x4i = _sc_gather(gi3, itR, b_per_w)
    qu = u32 & 3
    qi = i32 & 3
    return _mlp(x4u, x4i, qu, qi, W1, b1, W2, b2, W3, b3)
